# Initial kernel scaffold; baseline (speedup 1.0000x reference)
import jax
import jax.numpy as jnp
from jax.experimental import pallas as pl


def kernel(x, edge_index, edge_attr, W_in, b_in, We, be, Wpre, bpre, Wpost, bpost, Wlin, blin, ln_g, ln_b, Wo1, bo1, Wo2, bo2, Wo3, bo3):
    raise NotImplementedError("write your pallas kernel here")


# speedup vs baseline: 10.3491x; 10.3491x over previous
import functools
import math

import jax
import jax.numpy as jnp
import numpy as np
from jax.experimental import pallas as pl
from jax.experimental.pallas import tpu as pltpu

AVG_LOG = float(np.mean(np.log(np.arange(1, 31, dtype=np.float64))))
NB = 256
F = 128
NEG = -3.4028235e38
POS = 3.4028235e38


def _gelu(v):
    return 0.5 * v * (1.0 + jax.lax.erf(v * np.float32(1.0 / math.sqrt(2.0))))


def _row8(b):
    return jnp.broadcast_to(b[None, :], (8, b.shape[0]))


def _mm_gelu_body(x_ref, w_ref, b_ref, o_ref):
    o_ref[...] = _gelu(
        jnp.dot(x_ref[...], w_ref[...], preferred_element_type=jnp.float32)
        + b_ref[0, :][None, :])


def _ab_body(h_ref, wa_ref, wb_ref, ba_ref, a_ref, b_ref):
    h = h_ref[...]
    a = jnp.dot(h, wa_ref[...], preferred_element_type=jnp.float32) + ba_ref[0, :][None, :]
    b = jnp.dot(h, wb_ref[...], preferred_element_type=jnp.float32)
    for c in range(4):
        a_ref[c] = a[:, c * F:(c + 1) * F]
    for c in range(8):
        b_ref[c] = b[:, c * 64:(c + 1) * 64]


def _node_post_body(h_ref, a_ref, s1_ref, s2_ref, mx_ref, mn_ref, deg_ref,
                    wx_ref, w123_ref, wlin_ref, blin_ref, lng_ref, lnb_ref,
                    o_ref):
    t = pl.program_id(1)
    h = h_ref[...]
    deg = deg_ref[...]
    degc = jnp.maximum(deg, 1.0)
    inv = 1.0 / degc
    logd = jnp.log(degc + 1.0)
    s_amp = logd * (1.0 / AVG_LOG)
    s_att = AVG_LOG / logd
    mask = deg > 0.0

    a = a_ref[0]
    s1 = s1_ref[0]
    s2 = s2_ref[0]
    seg_mx = mx_ref[0]
    seg_mn = mn_ref[0]

    s = deg * a + s1
    mean = s * inv
    mx = jnp.where(mask, a + seg_mx, 0.0)
    mn = jnp.where(mask, a + seg_mn, 0.0)
    mean_sq = (deg * a * a + 2.0 * a * s1 + s2) * inv
    var = mean_sq - mean * mean
    std = jnp.sqrt(jnp.maximum(var, 0.0) + 1e-5)

    agg0 = jnp.concatenate([mean, mx, mn, std, var], axis=-1)
    z = jnp.dot(agg0, w123_ref[0], preferred_element_type=jnp.float32)
    out_t = (z[:, 0:32] + s_amp[:, 0:32] * z[:, 32:64]
             + s_att[:, 0:32] * z[:, 64:96])

    @pl.when(t == 0)
    def _():
        o_ref[...] = jnp.dot(h, wx_ref[...], preferred_element_type=jnp.float32)

    for tc in range(4):
        @pl.when(t == tc)
        def _(tc=tc):
            o_ref[:, tc * 32:(tc + 1) * 32] += out_t

    @pl.when(t == 3)
    def _():
        o = o_ref[...]
        o = jnp.dot(o, wlin_ref[...], preferred_element_type=jnp.float32) + blin_ref[0, :][None, :]
        mu = jnp.mean(o, axis=-1, keepdims=True)
        va = jnp.mean((o - mu) * (o - mu), axis=-1, keepdims=True)
        o = (o - mu) / jnp.sqrt(va + 1e-5) * lng_ref[0, :][None, :] + lnb_ref[0, :][None, :]
        o_ref[...] = _gelu(o) + h


def _out_mlp_body(h_ref, w1_ref, b1_ref, w2_ref, b2_ref, w3_ref, b3_ref, o_ref):
    o = _gelu(jnp.dot(h_ref[...], w1_ref[...], preferred_element_type=jnp.float32)
              + b1_ref[0, :][None, :])
    o = _gelu(jnp.dot(o, w2_ref[...], preferred_element_type=jnp.float32)
              + b2_ref[0, :][None, :])
    o_ref[...] = (jnp.dot(o, w3_ref[...], preferred_element_type=jnp.float32)
                  + b3_ref[0, :][None, :])


def _mm_gelu(x, w, b, np_):
    g = np_ // NB
    return pl.pallas_call(
        _mm_gelu_body,
        grid=(g,),
        in_specs=[
            pl.BlockSpec((NB, x.shape[1]), lambda i: (i, 0)),
            pl.BlockSpec(w.shape, lambda i: (0, 0)),
            pl.BlockSpec((8, b.shape[1]), lambda i: (0, 0)),
        ],
        out_specs=pl.BlockSpec((NB, w.shape[1]), lambda i: (i, 0)),
        out_shape=jax.ShapeDtypeStruct((np_, w.shape[1]), jnp.float32),
    )(x, w, b)


def _compute_ab(h, wa, wb, ba, np_):
    g = np_ // NB
    return pl.pallas_call(
        _ab_body,
        grid=(g,),
        in_specs=[
            pl.BlockSpec((NB, F), lambda i: (i, 0)),
            pl.BlockSpec((F, 512), lambda i: (0, 0)),
            pl.BlockSpec((F, 512), lambda i: (0, 0)),
            pl.BlockSpec((8, 512), lambda i: (0, 0)),
        ],
        out_specs=[
            pl.BlockSpec((4, NB, F), lambda i: (0, i, 0)),
            pl.BlockSpec((8, NB, 64), lambda i: (0, i, 0)),
        ],
        out_shape=[
            jax.ShapeDtypeStruct((4, np_, F), jnp.float32),
            jax.ShapeDtypeStruct((8, np_, 64), jnp.float32),
        ],
    )(h, wa, wb, ba)


def _node_post(h, a4, s1, s2, mx, mn, deg_b, wx, w123, wlin, blinp, lng, lnb, np_):
    g = np_ // NB
    chunk = lambda i, t: (t, i, 0)
    return pl.pallas_call(
        _node_post_body,
        grid=(g, 4),
        in_specs=[
            pl.BlockSpec((NB, F), lambda i, t: (i, 0)),
            pl.BlockSpec((1, NB, F), chunk),
            pl.BlockSpec((1, NB, F), chunk),
            pl.BlockSpec((1, NB, F), chunk),
            pl.BlockSpec((1, NB, F), chunk),
            pl.BlockSpec((1, NB, F), chunk),
            pl.BlockSpec((NB, F), lambda i, t: (i, 0)),
            pl.BlockSpec((F, F), lambda i, t: (0, 0)),
            pl.BlockSpec((1, 640, F), lambda i, t: (t, 0, 0)),
            pl.BlockSpec((F, F), lambda i, t: (0, 0)),
            pl.BlockSpec((8, F), lambda i, t: (0, 0)),
            pl.BlockSpec((8, F), lambda i, t: (0, 0)),
            pl.BlockSpec((8, F), lambda i, t: (0, 0)),
        ],
        out_specs=pl.BlockSpec((NB, F), lambda i, t: (i, 0)),
        out_shape=jax.ShapeDtypeStruct((np_, F), jnp.float32),
    )(h, a4, s1, s2, mx, mn, deg_b, wx, w123, wlin, blinp, lng, lnb)


def _out_mlp(h, w1, b1, w2, b2, w3, b3, np_):
    g = np_ // NB
    return pl.pallas_call(
        _out_mlp_body,
        grid=(g,),
        in_specs=[pl.BlockSpec((NB, F), lambda i: (i, 0))] + [
            spec for _ in range(3) for spec in (
                pl.BlockSpec((F, F), lambda i: (0, 0)),
                pl.BlockSpec((8, F), lambda i: (0, 0)),
            )
        ],
        out_specs=pl.BlockSpec((NB, F), lambda i: (i, 0)),
        out_shape=jax.ShapeDtypeStruct((np_, F), jnp.float32),
    )(h, w1, b1, w2, b2, w3, b3)


def _pad_w(w, rows=F, cols=F):
    out = jnp.zeros((rows, cols), jnp.float32)
    return out.at[:w.shape[0], :w.shape[1]].set(w)


def _pad_b(b, cols=F):
    out = jnp.zeros((cols,), jnp.float32)
    return out.at[:b.shape[0]].set(b)


def kernel(x, edge_index, edge_attr, W_in, b_in, We, be, Wpre, bpre, Wpost,
           bpost, Wlin, blin, ln_g, ln_b, Wo1, bo1, Wo2, bo2, Wo3, bo3):
    n = x.shape[0]
    e = edge_index.shape[1]
    L, T = Wpre.shape[0], Wpre.shape[1]
    np_ = ((n + NB - 1) // NB) * NB

    src = edge_index[0]
    dst = edge_index[1]

    xp = jnp.zeros((np_, x.shape[1]), jnp.float32).at[:n].set(x)
    h = _mm_gelu(xp, W_in, _row8(b_in), np_)

    deg = jax.ops.segment_sum(jnp.ones((e,), jnp.float32), dst, num_segments=n)
    degp = jnp.zeros((np_,), jnp.float32).at[:n].set(deg)
    deg_b = jnp.broadcast_to(degp[:, None], (np_, F))

    for l in range(L):
        Wflat = Wpre[l].transpose(1, 0, 2).reshape(3 * F, T * F)
        bflat = bpre[l].reshape(T * F)
        WA = Wflat[:F]
        WB = Wflat[F:2 * F]
        M = We[l] @ Wflat[2 * F:]
        aflat = bflat + be[l] @ Wflat[2 * F:]

        a4, b8 = _compute_ab(h, WA, WB, _row8(aflat), np_)

        B = b8.transpose(1, 0, 2).reshape(np_, 512)
        C = edge_attr @ M
        u = B[src] + C
        S1 = jax.ops.segment_sum(u, dst, num_segments=n)
        S2 = jax.ops.segment_sum(u * u, dst, num_segments=n)
        MX = jax.ops.segment_max(u, dst, num_segments=n)
        MX = jnp.where(jnp.isfinite(MX), MX, 0.0)
        MN = jax.ops.segment_min(u, dst, num_segments=n)
        MN = jnp.where(jnp.isfinite(MN), MN, 0.0)

        def chunk4(v):
            vp = jnp.zeros((np_, 512), jnp.float32).at[:n].set(v)
            return vp.reshape(np_, 4, F).transpose(1, 0, 2)
        s1c, s2c, mxc, mnc = chunk4(S1), chunk4(S2), chunk4(MX), chunk4(MN)

        WpX = jnp.concatenate([Wpost[l, t, :F, :] for t in range(T)], axis=1)
        W123 = jnp.stack([
            jnp.pad(jnp.concatenate([Wpost[l, t, F:F + 640, :],
                                     Wpost[l, t, F + 640:F + 1280, :],
                                     Wpost[l, t, F + 1280:, :]], axis=1),
                    ((0, 0), (0, 32)))
            for t in range(T)])
        bpost_flat = bpost[l].reshape(-1)
        blinp = bpost_flat @ Wlin[l] + blin[l]

        h = _node_post(h, a4, s1c, s2c, mxc, mnc, deg_b, WpX, W123, Wlin[l],
                       _row8(blinp), _row8(ln_g[l]), _row8(ln_b[l]), np_)

    o = _out_mlp(h, _pad_w(Wo1), _row8(_pad_b(bo1)),
                 _pad_w(Wo2), _row8(_pad_b(bo2)),
                 _pad_w(Wo3), _row8(_pad_b(bo3)), np_)
    return o[:n, 0]


# speedup vs baseline: 48.0507x; 4.6430x over previous
import dataclasses
import functools
import math

import jax
import jax.numpy as jnp
import numpy as np
from jax import lax
from jax.experimental import pallas as pl
from jax.experimental.pallas import tpu as pltpu
from jax.experimental.pallas import tpu_sc as plsc

AVG_LOG = float(np.mean(np.log(np.arange(1, 31, dtype=np.float64))))
NB = 256
F = 128
NEG = -3.4028235e38
POS = 3.4028235e38

NW = 32
NPW = 320
HPW = 160
NV = 64
CAP2 = 3328
ETILE = 2000
GB = 32
OT = 176

_mesh = plsc.VectorSubcoreMesh(core_axis_name="c", subcore_axis_name="s")

_sc_params = pltpu.CompilerParams()
if "needs_layout_passes" in pltpu.CompilerParams.__dataclass_fields__:
    _sc_params = dataclasses.replace(_sc_params, needs_layout_passes=False)


def _gelu(v):
    return 0.5 * v * (1.0 + jax.lax.erf(v * np.float32(1.0 / math.sqrt(2.0))))


def _row8(b):
    return jnp.broadcast_to(b[None, :], (8, b.shape[0]))


def _mm_gelu_body(x_ref, w_ref, b_ref, o_ref):
    o_ref[...] = _gelu(
        jnp.dot(x_ref[...], w_ref[...], preferred_element_type=jnp.float32)
        + b_ref[0, :][None, :])


def _ab_body(h_ref, wa_ref, wb_ref, ba_ref, a_ref, *b_refs):
    h = h_ref[...]
    a = jnp.dot(h, wa_ref[...], preferred_element_type=jnp.float32) + ba_ref[0, :][None, :]
    b = jnp.dot(h, wb_ref[...], preferred_element_type=jnp.float32)
    for c in range(4):
        a_ref[c] = a[:, c * F:(c + 1) * F]
    for c in range(4):
        b_refs[c][...] = b[:, c * F:(c + 1) * F]


def _c_body(ea_ref, m_ref, *c_refs):
    cc = jnp.dot(ea_ref[...], m_ref[...], preferred_element_type=jnp.float32)
    for c in range(4):
        c_refs[c][...] = cc[:, c * F:(c + 1) * F]


def _node_post_body(h_ref, a_ref, s1_ref, s2_ref, mx_ref, mn_ref, deg_ref,
                    wx_ref, w123_ref, wlin_ref, blin_ref, lng_ref, lnb_ref,
                    o_ref):
    t = pl.program_id(1)
    h = h_ref[...]
    deg = deg_ref[...]
    degc = jnp.maximum(deg, 1.0)
    inv = 1.0 / degc
    logd = jnp.log(degc + 1.0)
    s_amp = logd * (1.0 / AVG_LOG)
    s_att = AVG_LOG / logd
    mask = deg > 0.0

    a = a_ref[0]
    s1 = s1_ref[0]
    s2 = s2_ref[0]
    seg_mx = mx_ref[0]
    seg_mn = mn_ref[0]

    s = deg * a + s1
    mean = s * inv
    mx = jnp.where(mask, a + seg_mx, 0.0)
    mn = jnp.where(mask, a + seg_mn, 0.0)
    mean_sq = (deg * a * a + 2.0 * a * s1 + s2) * inv
    var = mean_sq - mean * mean
    std = jnp.sqrt(jnp.maximum(var, 0.0) + 1e-5)

    agg0 = jnp.concatenate([mean, mx, mn, std, var], axis=-1)
    z = jnp.dot(agg0, w123_ref[0], preferred_element_type=jnp.float32)
    out_t = (z[:, 0:32] + s_amp[:, 0:32] * z[:, 32:64]
             + s_att[:, 0:32] * z[:, 64:96])

    @pl.when(t == 0)
    def _():
        o_ref[...] = jnp.dot(h, wx_ref[...], preferred_element_type=jnp.float32)

    for tc in range(4):
        @pl.when(t == tc)
        def _(tc=tc):
            o_ref[:, tc * 32:(tc + 1) * 32] += out_t

    @pl.when(t == 3)
    def _():
        o = o_ref[...]
        o = jnp.dot(o, wlin_ref[...], preferred_element_type=jnp.float32) + blin_ref[0, :][None, :]
        mu = jnp.mean(o, axis=-1, keepdims=True)
        va = jnp.mean((o - mu) * (o - mu), axis=-1, keepdims=True)
        o = (o - mu) / jnp.sqrt(va + 1e-5) * lng_ref[0, :][None, :] + lnb_ref[0, :][None, :]
        o_ref[...] = _gelu(o) + h


def _out_mlp_body(h_ref, w1_ref, b1_ref, w2_ref, b2_ref, w3_ref, b3_ref, o_ref):
    o = _gelu(jnp.dot(h_ref[...], w1_ref[...], preferred_element_type=jnp.float32)
              + b1_ref[0, :][None, :])
    o = _gelu(jnp.dot(o, w2_ref[...], preferred_element_type=jnp.float32)
              + b2_ref[0, :][None, :])
    o_ref[...] = (jnp.dot(o, w3_ref[...], preferred_element_type=jnp.float32)
                  + b3_ref[0, :][None, :])


def _mm_gelu(x, w, b, np_):
    g = np_ // NB
    return pl.pallas_call(
        _mm_gelu_body,
        grid=(g,),
        in_specs=[
            pl.BlockSpec((NB, x.shape[1]), lambda i: (i, 0)),
            pl.BlockSpec(w.shape, lambda i: (0, 0)),
            pl.BlockSpec((8, b.shape[1]), lambda i: (0, 0)),
        ],
        out_specs=pl.BlockSpec((NB, w.shape[1]), lambda i: (i, 0)),
        out_shape=jax.ShapeDtypeStruct((np_, w.shape[1]), jnp.float32),
    )(x, w, b)


def _compute_ab(h, wa, wb, ba, np_):
    g = np_ // NB
    return pl.pallas_call(
        _ab_body,
        grid=(g,),
        in_specs=[
            pl.BlockSpec((NB, F), lambda i: (i, 0)),
            pl.BlockSpec((F, 512), lambda i: (0, 0)),
            pl.BlockSpec((F, 512), lambda i: (0, 0)),
            pl.BlockSpec((8, 512), lambda i: (0, 0)),
        ],
        out_specs=[pl.BlockSpec((4, NB, F), lambda i: (0, i, 0))] + [
            pl.BlockSpec((NB, F), lambda i: (i, 0)) for _ in range(4)],
        out_shape=[jax.ShapeDtypeStruct((4, np_, F), jnp.float32)] + [
            jax.ShapeDtypeStruct((np_, F), jnp.float32) for _ in range(4)],
    )(h, wa, wb, ba)


def _compute_c(eap, mp, e):
    eb = 2000
    g = e // eb
    return pl.pallas_call(
        _c_body,
        grid=(g,),
        in_specs=[
            pl.BlockSpec((eb, F), lambda i: (i, 0)),
            pl.BlockSpec((F, 512), lambda i: (0, 0)),
        ],
        out_specs=[pl.BlockSpec((eb, F), lambda i: (i, 0)) for _ in range(4)],
        out_shape=[jax.ShapeDtypeStruct((e, F), jnp.float32) for _ in range(4)],
    )(eap, mp)


def _node_post(h, a4, s1, s2, mx, mn, deg_b, wx, w123, wlin, blinp, lng, lnb, np_):
    g = np_ // NB
    chunk = lambda i, t: (t, i, 0)
    return pl.pallas_call(
        _node_post_body,
        grid=(g, 4),
        in_specs=[
            pl.BlockSpec((NB, F), lambda i, t: (i, 0)),
            pl.BlockSpec((1, NB, F), chunk),
            pl.BlockSpec((1, NB, F), chunk),
            pl.BlockSpec((1, NB, F), chunk),
            pl.BlockSpec((1, NB, F), chunk),
            pl.BlockSpec((1, NB, F), chunk),
            pl.BlockSpec((NB, F), lambda i, t: (i, 0)),
            pl.BlockSpec((F, F), lambda i, t: (0, 0)),
            pl.BlockSpec((1, 640, F), lambda i, t: (t, 0, 0)),
            pl.BlockSpec((F, F), lambda i, t: (0, 0)),
            pl.BlockSpec((8, F), lambda i, t: (0, 0)),
            pl.BlockSpec((8, F), lambda i, t: (0, 0)),
            pl.BlockSpec((8, F), lambda i, t: (0, 0)),
        ],
        out_specs=pl.BlockSpec((NB, F), lambda i, t: (i, 0)),
        out_shape=jax.ShapeDtypeStruct((np_, F), jnp.float32),
    )(h, a4, s1, s2, mx, mn, deg_b, wx, w123, wlin, blinp, lng, lnb)


def _out_mlp(h, w1, b1, w2, b2, w3, b3, np_):
    g = np_ // NB
    return pl.pallas_call(
        _out_mlp_body,
        grid=(g,),
        in_specs=[pl.BlockSpec((NB, F), lambda i: (i, 0))] + [
            spec for _ in range(3) for spec in (
                pl.BlockSpec((F, F), lambda i: (0, 0)),
                pl.BlockSpec((8, F), lambda i: (0, 0)),
            )
        ],
        out_specs=pl.BlockSpec((NB, F), lambda i: (i, 0)),
        out_shape=jax.ShapeDtypeStruct((np_, F), jnp.float32),
    )(h, w1, b1, w2, b2, w3, b3)


def _pad_w(w, rows=F, cols=F):
    out = jnp.zeros((rows, cols), jnp.float32)
    return out.at[:w.shape[0], :w.shape[1]].set(w)


def _pad_b(b, cols=F):
    out = jnp.zeros((cols,), jnp.float32)
    return out.at[:b.shape[0]].set(b)


def _sc_wid():
    return lax.axis_index("s") * 2 + lax.axis_index("c")


def _dma_wait(src, dst, sem):
    pltpu.make_async_copy(src, dst, sem).wait()


def _prep_body(dsta, srca, recs, offso,
               db0, sb0, db1, sb1,
               ld_u, src_u, eid_u, ld_s, src_s, eid_s, perm,
               hist, offs, offstg,
               sem0, sem1):
    w = _sc_wid()
    e = dsta.shape[0]
    ntiles = e // ETILE
    iota16 = lax.iota(jnp.int32, 16)
    m0 = iota16 == 0
    npw16 = jnp.full((16,), HPW, jnp.int32)
    zero16i = jnp.zeros((16,), jnp.int32)

    for v in range(2):
        vw = w * 2 + v
        lo = vw * HPW
        hi = lo + HPW

        pltpu.async_copy(dsta.at[pl.ds(0, ETILE)], db0, sem0)
        pltpu.async_copy(srca.at[pl.ds(0, ETILE)], sb0, sem0)
        pltpu.async_copy(dsta.at[pl.ds(ETILE, ETILE)], db1, sem1)
        pltpu.async_copy(srca.at[pl.ds(ETILE, ETILE)], sb1, sem1)

        def tilework(t, db, sb, sem, off):
            _dma_wait(dsta.at[pl.ds(0, ETILE)], db, sem)
            _dma_wait(srca.at[pl.ds(0, ETILE)], sb, sem)

            def vec(j, off):
                d = db[pl.ds(j * 16, 16)]
                s = sb[pl.ds(j * 16, 16)]
                m = (d >= lo) & (d < hi)
                mi = jnp.where(m, 1, 0).astype(jnp.int32)
                pre = plsc.cumsum(mi)
                cnt = jnp.sum(mi)
                offc = jnp.minimum(off, CAP2 - 17)
                pos = offc + pre - 1
                plsc.store_scatter(ld_u, [pos], d - lo, mask=m)
                plsc.store_scatter(src_u, [pos], s, mask=m)
                eidv = (t * ETILE + j * 16) + iota16
                plsc.store_scatter(eid_u, [pos], eidv, mask=m)
                return off + cnt

            off = lax.fori_loop(0, ETILE // 16, vec, off)

            @pl.when(t + 2 < ntiles)
            def _():
                pltpu.async_copy(dsta.at[pl.ds((t + 2) * ETILE, ETILE)], db, sem)
                pltpu.async_copy(srca.at[pl.ds((t + 2) * ETILE, ETILE)], sb, sem)
            return off

        def outer(k, off):
            off = tilework(2 * k, db0, sb0, sem0, off)
            off = tilework(2 * k + 1, db1, sb1, sem1, off)
            return off

        cnt = lax.fori_loop(0, ntiles // 2, outer, jnp.int32(0))
        cnt = jnp.minimum(cnt, CAP2 - 128)
        cnt128 = ((cnt + 127) // 128) * 128

        for j in range(8):
            idxv = cnt + j * 16 + iota16
            mpad = idxv < cnt128
            plsc.store_scatter(ld_u, [idxv], npw16, mask=mpad)
            plsc.store_scatter(src_u, [idxv], zero16i, mask=mpad)
            plsc.store_scatter(eid_u, [idxv], zero16i, mask=mpad)

        def zeroh(i, _):
            hist[i] = jnp.int32(0)
            return 0
        lax.fori_loop(0, OT, zeroh, 0)

        def histb(j, _):
            lv = ld_u[pl.ds(j * 16, 16)]
            for i in range(16):
                l = lv[i]
                hist[l] = hist[l] + 1
            return 0
        lax.fori_loop(0, cnt128 // 16, histb, 0)

        def cumul(b, run):
            hv = hist[b]
            offs[b] = run
            plsc.store_scatter(offstg, [jnp.broadcast_to(b, (16,))],
                               jnp.broadcast_to(run, (16,)), mask=m0)
            return run + hv
        lax.fori_loop(0, OT, cumul, jnp.int32(0))

        def place(j, _):
            lv = ld_u[pl.ds(j * 16, 16)]
            for i in range(16):
                l = lv[i]
                p = offs[l]
                offs[l] = p + 1
                plsc.store_scatter(perm, [jnp.broadcast_to(p, (16,))],
                                   jnp.broadcast_to(j * 16 + i, (16,)), mask=m0)
            return 0
        lax.fori_loop(0, cnt128 // 16, place, 0)

        def apply(j, _):
            idx = perm[pl.ds(j * 16, 16)]
            ld_s[pl.ds(j * 16, 16)] = plsc.load_gather(ld_u, [idx])
            src_s[pl.ds(j * 16, 16)] = plsc.load_gather(src_u, [idx])
            eid_s[pl.ds(j * 16, 16)] = plsc.load_gather(eid_u, [idx])
            return 0
        lax.fori_loop(0, cnt128 // 16, apply, 0)

        pltpu.sync_copy(ld_s, recs.at[pl.ds((0 * NV + vw) * CAP2, CAP2)])
        pltpu.sync_copy(src_s, recs.at[pl.ds((1 * NV + vw) * CAP2, CAP2)])
        pltpu.sync_copy(eid_s, recs.at[pl.ds((2 * NV + vw) * CAP2, CAP2)])
        pltpu.sync_copy(offstg, offso.at[pl.ds(vw * OT, OT)])


def _sc_prep(dsta, srca):
    fn = pl.kernel(
        _prep_body,
        out_type=[
            jax.ShapeDtypeStruct((3 * NV * CAP2,), jnp.int32),
            jax.ShapeDtypeStruct((NV * OT,), jnp.int32),
        ],
        mesh=_mesh,
        scratch_types=[
            pltpu.VMEM((ETILE,), jnp.int32),
            pltpu.VMEM((ETILE,), jnp.int32),
            pltpu.VMEM((ETILE,), jnp.int32),
            pltpu.VMEM((ETILE,), jnp.int32),
            pltpu.VMEM((CAP2,), jnp.int32),
            pltpu.VMEM((CAP2,), jnp.int32),
            pltpu.VMEM((CAP2,), jnp.int32),
            pltpu.VMEM((CAP2,), jnp.int32),
            pltpu.VMEM((CAP2,), jnp.int32),
            pltpu.VMEM((CAP2,), jnp.int32),
            pltpu.VMEM((CAP2,), jnp.int32),
            pltpu.SMEM((OT,), jnp.int32),
            pltpu.SMEM((OT,), jnp.int32),
            pltpu.VMEM((OT,), jnp.int32),
            pltpu.SemaphoreType.DMA,
            pltpu.SemaphoreType.DMA,
        ],
        compiler_params=_sc_params,
    )
    return fn(dsta, srca)


def _stats_body(recs, offsi, bs, cs, s1o, s2o, mxo, mno,
                srcr, eidr, bb0, bc0, bb1, bc1,
                stg1, stg2, stgx, stgn, offv, offm,
                semb0, semc0, semb1, semc1):
    w = _sc_wid()
    zero = jnp.zeros((16,), jnp.float32)
    neg = jnp.full((16,), NEG, jnp.float32)
    pos = jnp.full((16,), POS, jnp.float32)

    for v in range(2):
        vw = 2 * w + v
        pltpu.sync_copy(recs.at[pl.ds((1 * NV + vw) * CAP2, CAP2)], srcr)
        pltpu.sync_copy(recs.at[pl.ds((2 * NV + vw) * CAP2, CAP2)], eidr)
        pltpu.sync_copy(offsi.at[pl.ds(vw * OT, OT)], offv)
        for j in range(OT // 16):
            ov = offv[pl.ds(j * 16, 16)]
            for i in range(16):
                offm[j * 16 + i] = ov[i]
        total = offm[HPW]
        cnt128 = ((total + 127) // 128) * 128
        nb = cnt128 // GB

        for t in range(4):
            bp = bs[t]
            cp = cs[t]

            def initrow(r, _):
                for k in range(8):
                    sl = pl.ds(r * F + k * 16, 16)
                    stg1[sl] = zero
                    stg2[sl] = zero
                    stgx[sl] = neg
                    stgn[sl] = pos
                return 0
            lax.fori_loop(0, HPW + 8, initrow, 0)

            def issue(g, bb, bc, semb, semc):
                idxs = srcr.at[pl.ds(g * GB, GB)]
                pltpu.async_copy(bp.at[idxs], bb, semb)
                idxe = eidr.at[pl.ds(g * GB, GB)]
                pltpu.async_copy(cp.at[idxe], bc, semc)

            @pl.when(nb > 0)
            def _():
                issue(0, bb0, bc0, semb0, semc0)

            @pl.when(nb > 1)
            def _():
                issue(1, bb1, bc1, semb1, semc1)

            def half(g, bb, bc, semb, semc, carry):
                _dma_wait(bp.at[pl.ds(0, GB)], bb, semb)
                _dma_wait(cp.at[pl.ds(0, GB)], bc, semc)
                end = jnp.minimum((g + 1) * GB, total)

                def wcond(st):
                    return st[1] < end

                def wbody(st):
                    n = st[0]
                    r = st[1]
                    acc = st[2:]
                    nxt = offm[n + 1]
                    stop = jnp.minimum(nxt, end)

                    def rec(i, acc):
                        ri = i - g * GB
                        out = []
                        for k in range(8):
                            sl = pl.ds(k * 16, 16)
                            u = bb[ri, sl] + bc[ri, sl]
                            out.append(acc[k] + u)
                            out.append(acc[8 + k] + u * u)
                            out.append(jnp.maximum(acc[16 + k], u))
                            out.append(jnp.minimum(acc[24 + k], u))
                        return tuple(out[0::4] + out[1::4] + out[2::4]
                                     + out[3::4])

                    acc = lax.fori_loop(r, stop, rec, tuple(acc))
                    done = stop == nxt

                    @pl.when(done)
                    def _(acc=acc, n=n):
                        for k in range(8):
                            sl = pl.ds(n * F + k * 16, 16)
                            stg1[sl] = acc[k]
                            stg2[sl] = acc[8 + k]
                            stgx[sl] = acc[16 + k]
                            stgn[sl] = acc[24 + k]

                    newacc = []
                    for k in range(8):
                        newacc.append(jnp.where(done, zero, acc[k]))
                    for k in range(8):
                        newacc.append(jnp.where(done, zero, acc[8 + k]))
                    for k in range(8):
                        newacc.append(jnp.where(done, neg, acc[16 + k]))
                    for k in range(8):
                        newacc.append(jnp.where(done, pos, acc[24 + k]))
                    return (n + jnp.where(done, 1, 0), stop) + tuple(newacc)

                st = lax.while_loop(wcond, wbody, carry)

                @pl.when(g + 2 < nb)
                def _():
                    issue(g + 2, bb, bc, semb, semc)
                return st

            carry0 = ((jnp.int32(0), jnp.int32(0)) + tuple([zero] * 8)
                      + tuple([zero] * 8) + tuple([neg] * 8)
                      + tuple([pos] * 8))

            def outer(k, carry):
                carry = half(2 * k, bb0, bc0, semb0, semc0, carry)
                carry = half(2 * k + 1, bb1, bc1, semb1, semc1, carry)
                return carry

            lax.fori_loop(0, nb // 2, outer, carry0)

            osl = pl.ds((t * NW * NPW + vw * HPW) * F, HPW * F)
            ssl = pl.ds(0, HPW * F)
            pltpu.sync_copy(stg1.at[ssl], s1o.at[osl])
            pltpu.sync_copy(stg2.at[ssl], s2o.at[osl])
            pltpu.sync_copy(stgx.at[ssl], mxo.at[osl])
            pltpu.sync_copy(stgn.at[ssl], mno.at[osl])


def _sc_stats(recs, offsi, bs, cs, np_):
    def body(recs, offsi, b0, b1, b2, b3, c0, c1, c2, c3,
             s1o, s2o, mxo, mno, *scratch):
        _stats_body(recs, offsi, (b0, b1, b2, b3), (c0, c1, c2, c3),
                    s1o, s2o, mxo, mno, *scratch)

    fn = pl.kernel(
        body,
        out_type=[jax.ShapeDtypeStruct((4 * np_ * F,), jnp.float32)
                  for _ in range(4)],
        mesh=_mesh,
        scratch_types=[
            pltpu.VMEM((CAP2,), jnp.int32),
            pltpu.VMEM((CAP2,), jnp.int32),
            pltpu.VMEM((GB, F), jnp.float32),
            pltpu.VMEM((GB, F), jnp.float32),
            pltpu.VMEM((GB, F), jnp.float32),
            pltpu.VMEM((GB, F), jnp.float32),
            pltpu.VMEM(((HPW + 8) * F,), jnp.float32),
            pltpu.VMEM(((HPW + 8) * F,), jnp.float32),
            pltpu.VMEM(((HPW + 8) * F,), jnp.float32),
            pltpu.VMEM(((HPW + 8) * F,), jnp.float32),
            pltpu.VMEM((OT,), jnp.int32),
            pltpu.SMEM((OT,), jnp.int32),
            pltpu.SemaphoreType.DMA,
            pltpu.SemaphoreType.DMA,
            pltpu.SemaphoreType.DMA,
            pltpu.SemaphoreType.DMA,
        ],
        compiler_params=_sc_params,
    )
    return fn(recs, offsi, *bs, *cs)


def kernel(x, edge_index, edge_attr, W_in, b_in, We, be, Wpre, bpre, Wpost,
           bpost, Wlin, blin, ln_g, ln_b, Wo1, bo1, Wo2, bo2, Wo3, bo3):
    n = x.shape[0]
    e = edge_index.shape[1]
    L, T = Wpre.shape[0], Wpre.shape[1]
    np_ = NW * NPW

    xp = jnp.zeros((np_, x.shape[1]), jnp.float32).at[:n].set(x)
    h = _mm_gelu(xp, W_in, _row8(b_in), np_)

    recs, offsa = _sc_prep(edge_index[1], edge_index[0])
    offs2 = offsa.reshape(NV, OT)
    deg = (offs2[:, 1:HPW + 1] - offs2[:, :HPW]).reshape(-1).astype(jnp.float32)
    deg_b = jnp.broadcast_to(deg[:, None], (np_, F))

    eap = jnp.pad(edge_attr, ((0, 0), (0, F - edge_attr.shape[1])))

    for l in range(L):
        Wflat = Wpre[l].transpose(1, 0, 2).reshape(3 * F, T * F)
        bflat = bpre[l].reshape(T * F)
        WA = Wflat[:F]
        WB = Wflat[F:2 * F]
        M = We[l] @ Wflat[2 * F:]
        aflat = bflat + be[l] @ Wflat[2 * F:]
        Mp = jnp.zeros((F, 512), jnp.float32).at[:4].set(M)

        ab = _compute_ab(h, WA, WB, _row8(aflat), np_)
        a4, bs = ab[0], ab[1:]
        cs = _compute_c(eap, Mp, e)

        s1f, s2f, mxf, mnf = _sc_stats(recs, offsa, bs, cs, np_)
        s1c = s1f.reshape(4, np_, F)
        s2c = s2f.reshape(4, np_, F)
        mxc = mxf.reshape(4, np_, F)
        mnc = mnf.reshape(4, np_, F)

        WpX = jnp.concatenate([Wpost[l, t, :F, :] for t in range(T)], axis=1)
        W123 = jnp.stack([
            jnp.pad(jnp.concatenate([Wpost[l, t, F:F + 640, :],
                                     Wpost[l, t, F + 640:F + 1280, :],
                                     Wpost[l, t, F + 1280:, :]], axis=1),
                    ((0, 0), (0, 32)))
            for t in range(T)])
        bpost_flat = bpost[l].reshape(-1)
        blinp = bpost_flat @ Wlin[l] + blin[l]

        h = _node_post(h, a4, s1c, s2c, mxc, mnc, deg_b, WpX, W123, Wlin[l],
                       _row8(blinp), _row8(ln_g[l]), _row8(ln_b[l]), np_)

    o = _out_mlp(h, _pad_w(Wo1), _row8(_pad_b(bo1)),
                 _pad_w(Wo2), _row8(_pad_b(bo2)),
                 _pad_w(Wo3), _row8(_pad_b(bo3)), np_)
    return o[:n, 0]


# speedup vs baseline: 52.4738x; 1.0921x over previous
import dataclasses
import functools
import math

import jax
import jax.numpy as jnp
import numpy as np
from jax import lax
from jax.experimental import pallas as pl
from jax.experimental.pallas import tpu as pltpu
from jax.experimental.pallas import tpu_sc as plsc

AVG_LOG = float(np.mean(np.log(np.arange(1, 31, dtype=np.float64))))
NB = 256
F = 128
NEG = -3.4028235e38
POS = 3.4028235e38

NW = 32
NPW = 320
HPW = 160
NV = 64
CAP2 = 3328
ETILE = 2000
GB = 64
OT = 176

_mesh = plsc.VectorSubcoreMesh(core_axis_name="c", subcore_axis_name="s")

_sc_params = pltpu.CompilerParams()
if "needs_layout_passes" in pltpu.CompilerParams.__dataclass_fields__:
    _sc_params = dataclasses.replace(_sc_params, needs_layout_passes=False)


def _gelu(v):
    return 0.5 * v * (1.0 + jax.lax.erf(v * np.float32(1.0 / math.sqrt(2.0))))


def _row8(b):
    return jnp.broadcast_to(b[None, :], (8, b.shape[0]))


def _mm_gelu_body(x_ref, w_ref, b_ref, o_ref):
    o_ref[...] = _gelu(
        jnp.dot(x_ref[...], w_ref[...], preferred_element_type=jnp.float32)
        + b_ref[0, :][None, :])


def _ab_body(h_ref, wa_ref, wb_ref, ba_ref, a_ref, *b_refs):
    h = h_ref[...]
    a = jnp.dot(h, wa_ref[...], preferred_element_type=jnp.float32) + ba_ref[0, :][None, :]
    b = jnp.dot(h, wb_ref[...], preferred_element_type=jnp.float32)
    for c in range(4):
        a_ref[c] = a[:, c * F:(c + 1) * F]
    for c in range(4):
        b_refs[c][...] = b[:, c * F:(c + 1) * F]


def _c_body(ea_ref, m_ref, *c_refs):
    cc = jnp.dot(ea_ref[...], m_ref[...], preferred_element_type=jnp.float32)
    for c in range(4):
        c_refs[c][...] = cc[:, c * F:(c + 1) * F]


def _node_post_body(h_ref, a_ref, s1_ref, s2_ref, mx_ref, mn_ref, deg_ref,
                    wx_ref, w123_ref, wlin_ref, blin_ref, lng_ref, lnb_ref,
                    o_ref):
    t = pl.program_id(1)
    h = h_ref[...]
    deg = deg_ref[...]
    degc = jnp.maximum(deg, 1.0)
    inv = 1.0 / degc
    logd = jnp.log(degc + 1.0)
    s_amp = logd * (1.0 / AVG_LOG)
    s_att = AVG_LOG / logd
    mask = deg > 0.0

    a = a_ref[0]
    s1 = s1_ref[0]
    s2 = s2_ref[0]
    seg_mx = mx_ref[0]
    seg_mn = mn_ref[0]

    s = deg * a + s1
    mean = s * inv
    mx = jnp.where(mask, a + seg_mx, 0.0)
    mn = jnp.where(mask, a + seg_mn, 0.0)
    mean_sq = (deg * a * a + 2.0 * a * s1 + s2) * inv
    var = mean_sq - mean * mean
    std = jnp.sqrt(jnp.maximum(var, 0.0) + 1e-5)

    agg0 = jnp.concatenate([mean, mx, mn, std, var], axis=-1)
    z = jnp.dot(agg0, w123_ref[0], preferred_element_type=jnp.float32)
    out_t = (z[:, 0:32] + s_amp[:, 0:32] * z[:, 32:64]
             + s_att[:, 0:32] * z[:, 64:96])

    @pl.when(t == 0)
    def _():
        o_ref[...] = jnp.dot(h, wx_ref[...], preferred_element_type=jnp.float32)

    for tc in range(4):
        @pl.when(t == tc)
        def _(tc=tc):
            o_ref[:, tc * 32:(tc + 1) * 32] += out_t

    @pl.when(t == 3)
    def _():
        o = o_ref[...]
        o = jnp.dot(o, wlin_ref[...], preferred_element_type=jnp.float32) + blin_ref[0, :][None, :]
        mu = jnp.mean(o, axis=-1, keepdims=True)
        va = jnp.mean((o - mu) * (o - mu), axis=-1, keepdims=True)
        o = (o - mu) / jnp.sqrt(va + 1e-5) * lng_ref[0, :][None, :] + lnb_ref[0, :][None, :]
        o_ref[...] = _gelu(o) + h


def _out_mlp_body(h_ref, w1_ref, b1_ref, w2_ref, b2_ref, w3_ref, b3_ref, o_ref):
    o = _gelu(jnp.dot(h_ref[...], w1_ref[...], preferred_element_type=jnp.float32)
              + b1_ref[0, :][None, :])
    o = _gelu(jnp.dot(o, w2_ref[...], preferred_element_type=jnp.float32)
              + b2_ref[0, :][None, :])
    o_ref[...] = (jnp.dot(o, w3_ref[...], preferred_element_type=jnp.float32)
                  + b3_ref[0, :][None, :])


def _mm_gelu(x, w, b, np_):
    g = np_ // NB
    return pl.pallas_call(
        _mm_gelu_body,
        grid=(g,),
        in_specs=[
            pl.BlockSpec((NB, x.shape[1]), lambda i: (i, 0)),
            pl.BlockSpec(w.shape, lambda i: (0, 0)),
            pl.BlockSpec((8, b.shape[1]), lambda i: (0, 0)),
        ],
        out_specs=pl.BlockSpec((NB, w.shape[1]), lambda i: (i, 0)),
        out_shape=jax.ShapeDtypeStruct((np_, w.shape[1]), jnp.float32),
    )(x, w, b)


def _compute_ab(h, wa, wb, ba, np_):
    g = np_ // NB
    return pl.pallas_call(
        _ab_body,
        grid=(g,),
        in_specs=[
            pl.BlockSpec((NB, F), lambda i: (i, 0)),
            pl.BlockSpec((F, 512), lambda i: (0, 0)),
            pl.BlockSpec((F, 512), lambda i: (0, 0)),
            pl.BlockSpec((8, 512), lambda i: (0, 0)),
        ],
        out_specs=[pl.BlockSpec((4, NB, F), lambda i: (0, i, 0))] + [
            pl.BlockSpec((NB, F), lambda i: (i, 0)) for _ in range(4)],
        out_shape=[jax.ShapeDtypeStruct((4, np_, F), jnp.float32)] + [
            jax.ShapeDtypeStruct((np_, F), jnp.float32) for _ in range(4)],
    )(h, wa, wb, ba)


def _compute_c(eap, mp, e):
    eb = 2000
    g = e // eb
    return pl.pallas_call(
        _c_body,
        grid=(g,),
        in_specs=[
            pl.BlockSpec((eb, F), lambda i: (i, 0)),
            pl.BlockSpec((F, 512), lambda i: (0, 0)),
        ],
        out_specs=[pl.BlockSpec((eb, F), lambda i: (i, 0)) for _ in range(4)],
        out_shape=[jax.ShapeDtypeStruct((e, F), jnp.float32) for _ in range(4)],
    )(eap, mp)


def _node_post(h, a4, s1, s2, mx, mn, deg_b, wx, w123, wlin, blinp, lng, lnb, np_):
    g = np_ // NB
    chunk = lambda i, t: (t, i, 0)
    return pl.pallas_call(
        _node_post_body,
        grid=(g, 4),
        in_specs=[
            pl.BlockSpec((NB, F), lambda i, t: (i, 0)),
            pl.BlockSpec((1, NB, F), chunk),
            pl.BlockSpec((1, NB, F), chunk),
            pl.BlockSpec((1, NB, F), chunk),
            pl.BlockSpec((1, NB, F), chunk),
            pl.BlockSpec((1, NB, F), chunk),
            pl.BlockSpec((NB, F), lambda i, t: (i, 0)),
            pl.BlockSpec((F, F), lambda i, t: (0, 0)),
            pl.BlockSpec((1, 640, F), lambda i, t: (t, 0, 0)),
            pl.BlockSpec((F, F), lambda i, t: (0, 0)),
            pl.BlockSpec((8, F), lambda i, t: (0, 0)),
            pl.BlockSpec((8, F), lambda i, t: (0, 0)),
            pl.BlockSpec((8, F), lambda i, t: (0, 0)),
        ],
        out_specs=pl.BlockSpec((NB, F), lambda i, t: (i, 0)),
        out_shape=jax.ShapeDtypeStruct((np_, F), jnp.float32),
    )(h, a4, s1, s2, mx, mn, deg_b, wx, w123, wlin, blinp, lng, lnb)


def _out_mlp(h, w1, b1, w2, b2, w3, b3, np_):
    g = np_ // NB
    return pl.pallas_call(
        _out_mlp_body,
        grid=(g,),
        in_specs=[pl.BlockSpec((NB, F), lambda i: (i, 0))] + [
            spec for _ in range(3) for spec in (
                pl.BlockSpec((F, F), lambda i: (0, 0)),
                pl.BlockSpec((8, F), lambda i: (0, 0)),
            )
        ],
        out_specs=pl.BlockSpec((NB, F), lambda i: (i, 0)),
        out_shape=jax.ShapeDtypeStruct((np_, F), jnp.float32),
    )(h, w1, b1, w2, b2, w3, b3)


def _pad_w(w, rows=F, cols=F):
    out = jnp.zeros((rows, cols), jnp.float32)
    return out.at[:w.shape[0], :w.shape[1]].set(w)


def _pad_b(b, cols=F):
    out = jnp.zeros((cols,), jnp.float32)
    return out.at[:b.shape[0]].set(b)


def _sc_wid():
    return lax.axis_index("s") * 2 + lax.axis_index("c")


def _dma_wait(src, dst, sem):
    pltpu.make_async_copy(src, dst, sem).wait()


def _prep_body(dsta, srca, recs, offso,
               db0, sb0, db1, sb1,
               ld_u, src_u, eid_u, ld_s, src_s, eid_s, perm,
               hist, offs, offstg,
               sem0, sem1):
    w = _sc_wid()
    e = dsta.shape[0]
    ntiles = e // ETILE
    iota16 = lax.iota(jnp.int32, 16)
    m0 = iota16 == 0
    npw16 = jnp.full((16,), HPW, jnp.int32)
    zero16i = jnp.zeros((16,), jnp.int32)

    for v in range(2):
        vw = w * 2 + v
        lo = vw * HPW
        hi = lo + HPW

        pltpu.async_copy(dsta.at[pl.ds(0, ETILE)], db0, sem0)
        pltpu.async_copy(srca.at[pl.ds(0, ETILE)], sb0, sem0)
        pltpu.async_copy(dsta.at[pl.ds(ETILE, ETILE)], db1, sem1)
        pltpu.async_copy(srca.at[pl.ds(ETILE, ETILE)], sb1, sem1)

        def tilework(t, db, sb, sem, off):
            _dma_wait(dsta.at[pl.ds(0, ETILE)], db, sem)
            _dma_wait(srca.at[pl.ds(0, ETILE)], sb, sem)

            def vec(j, off):
                d = db[pl.ds(j * 16, 16)]
                s = sb[pl.ds(j * 16, 16)]
                m = (d >= lo) & (d < hi)
                mi = jnp.where(m, 1, 0).astype(jnp.int32)
                pre = plsc.cumsum(mi)
                cnt = jnp.sum(mi)
                offc = jnp.minimum(off, CAP2 - 17)
                pos = offc + pre - 1
                plsc.store_scatter(ld_u, [pos], d - lo, mask=m)
                plsc.store_scatter(src_u, [pos], s, mask=m)
                eidv = (t * ETILE + j * 16) + iota16
                plsc.store_scatter(eid_u, [pos], eidv, mask=m)
                return off + cnt

            off = lax.fori_loop(0, ETILE // 16, vec, off)

            @pl.when(t + 2 < ntiles)
            def _():
                pltpu.async_copy(dsta.at[pl.ds((t + 2) * ETILE, ETILE)], db, sem)
                pltpu.async_copy(srca.at[pl.ds((t + 2) * ETILE, ETILE)], sb, sem)
            return off

        def outer(k, off):
            off = tilework(2 * k, db0, sb0, sem0, off)
            off = tilework(2 * k + 1, db1, sb1, sem1, off)
            return off

        cnt = lax.fori_loop(0, ntiles // 2, outer, jnp.int32(0))
        cnt = jnp.minimum(cnt, CAP2 - 128)
        cnt128 = ((cnt + 127) // 128) * 128

        for j in range(8):
            idxv = cnt + j * 16 + iota16
            mpad = idxv < cnt128
            plsc.store_scatter(ld_u, [idxv], npw16, mask=mpad)
            plsc.store_scatter(src_u, [idxv], zero16i, mask=mpad)
            plsc.store_scatter(eid_u, [idxv], zero16i, mask=mpad)

        def zeroh(i, _):
            hist[i] = jnp.int32(0)
            return 0
        lax.fori_loop(0, OT, zeroh, 0)

        def histb(j, _):
            lv = ld_u[pl.ds(j * 16, 16)]
            for i in range(16):
                l = lv[i]
                hist[l] = hist[l] + 1
            return 0
        lax.fori_loop(0, cnt128 // 16, histb, 0)

        def cumul(b, run):
            hv = hist[b]
            offs[b] = run
            plsc.store_scatter(offstg, [jnp.broadcast_to(b, (16,))],
                               jnp.broadcast_to(run, (16,)), mask=m0)
            return run + hv
        lax.fori_loop(0, OT, cumul, jnp.int32(0))

        def place(j, _):
            lv = ld_u[pl.ds(j * 16, 16)]
            for i in range(16):
                l = lv[i]
                p = offs[l]
                offs[l] = p + 1
                plsc.store_scatter(perm, [jnp.broadcast_to(p, (16,))],
                                   jnp.broadcast_to(j * 16 + i, (16,)), mask=m0)
            return 0
        lax.fori_loop(0, cnt128 // 16, place, 0)

        def apply(j, _):
            idx = perm[pl.ds(j * 16, 16)]
            ld_s[pl.ds(j * 16, 16)] = plsc.load_gather(ld_u, [idx])
            src_s[pl.ds(j * 16, 16)] = plsc.load_gather(src_u, [idx])
            eid_s[pl.ds(j * 16, 16)] = plsc.load_gather(eid_u, [idx])
            return 0
        lax.fori_loop(0, cnt128 // 16, apply, 0)

        pltpu.sync_copy(ld_s, recs.at[pl.ds((0 * NV + vw) * CAP2, CAP2)])
        pltpu.sync_copy(src_s, recs.at[pl.ds((1 * NV + vw) * CAP2, CAP2)])
        pltpu.sync_copy(eid_s, recs.at[pl.ds((2 * NV + vw) * CAP2, CAP2)])
        pltpu.sync_copy(offstg, offso.at[pl.ds(vw * OT, OT)])


def _sc_prep(dsta, srca):
    fn = pl.kernel(
        _prep_body,
        out_type=[
            jax.ShapeDtypeStruct((3 * NV * CAP2,), jnp.int32),
            jax.ShapeDtypeStruct((NV * OT,), jnp.int32),
        ],
        mesh=_mesh,
        scratch_types=[
            pltpu.VMEM((ETILE,), jnp.int32),
            pltpu.VMEM((ETILE,), jnp.int32),
            pltpu.VMEM((ETILE,), jnp.int32),
            pltpu.VMEM((ETILE,), jnp.int32),
            pltpu.VMEM((CAP2,), jnp.int32),
            pltpu.VMEM((CAP2,), jnp.int32),
            pltpu.VMEM((CAP2,), jnp.int32),
            pltpu.VMEM((CAP2,), jnp.int32),
            pltpu.VMEM((CAP2,), jnp.int32),
            pltpu.VMEM((CAP2,), jnp.int32),
            pltpu.VMEM((CAP2,), jnp.int32),
            pltpu.SMEM((OT,), jnp.int32),
            pltpu.SMEM((OT,), jnp.int32),
            pltpu.VMEM((OT,), jnp.int32),
            pltpu.SemaphoreType.DMA,
            pltpu.SemaphoreType.DMA,
        ],
        compiler_params=_sc_params,
    )
    return fn(dsta, srca)


def _stats_body(recs, offsi, bs, cs, s1o, s2o, mxo, mno,
                srcr, eidr, bb0, bc0, bb1, bc1,
                stg1, stg2, stgx, stgn, offv, offm,
                semb0, semc0, semb1, semc1):
    w = _sc_wid()
    zero = jnp.zeros((16,), jnp.float32)
    neg = jnp.full((16,), NEG, jnp.float32)
    pos = jnp.full((16,), POS, jnp.float32)

    for v in range(2):
        vw = 2 * w + v
        pltpu.sync_copy(recs.at[pl.ds((1 * NV + vw) * CAP2, CAP2)], srcr)
        pltpu.sync_copy(recs.at[pl.ds((2 * NV + vw) * CAP2, CAP2)], eidr)
        pltpu.sync_copy(offsi.at[pl.ds(vw * OT, OT)], offv)
        for j in range(OT // 16):
            ov = offv[pl.ds(j * 16, 16)]
            for i in range(16):
                offm[j * 16 + i] = ov[i]
        total = offm[HPW]
        cnt128 = ((total + 127) // 128) * 128
        nb = cnt128 // GB

        for t in range(4):
            bp = bs[t]
            cp = cs[t]

            def initrow(r, _):
                for k in range(8):
                    sl = pl.ds(r * F + k * 16, 16)
                    stg1[sl] = zero
                    stg2[sl] = zero
                    stgx[sl] = neg
                    stgn[sl] = pos
                return 0
            lax.fori_loop(0, HPW + 8, initrow, 0)

            def issue(g, bb, bc, semb, semc):
                idxs = srcr.at[pl.ds(g * GB, GB)]
                pltpu.async_copy(bp.at[idxs], bb, semb)
                idxe = eidr.at[pl.ds(g * GB, GB)]
                pltpu.async_copy(cp.at[idxe], bc, semc)

            @pl.when(nb > 0)
            def _():
                issue(0, bb0, bc0, semb0, semc0)

            @pl.when(nb > 1)
            def _():
                issue(1, bb1, bc1, semb1, semc1)

            def half(g, bb, bc, semb, semc, carry):
                _dma_wait(bp.at[pl.ds(0, GB)], bb, semb)
                _dma_wait(cp.at[pl.ds(0, GB)], bc, semc)
                end = jnp.minimum((g + 1) * GB, total)

                def wcond(st):
                    return st[1] < end

                def wbody(st):
                    n = st[0]
                    r = st[1]
                    acc = st[2:]
                    nxt = offm[n + 1]
                    stop = jnp.minimum(nxt, end)

                    def rec(i, acc):
                        ri = i - g * GB
                        out = []
                        for k in range(8):
                            sl = pl.ds(k * 16, 16)
                            u = bb[ri, sl] + bc[ri, sl]
                            out.append(acc[k] + u)
                            out.append(acc[8 + k] + u * u)
                            out.append(jnp.maximum(acc[16 + k], u))
                            out.append(jnp.minimum(acc[24 + k], u))
                        return tuple(out[0::4] + out[1::4] + out[2::4]
                                     + out[3::4])

                    acc = lax.fori_loop(r, stop, rec, tuple(acc))
                    done = stop == nxt

                    @pl.when(done)
                    def _(acc=acc, n=n):
                        for k in range(8):
                            sl = pl.ds(n * F + k * 16, 16)
                            stg1[sl] = acc[k]
                            stg2[sl] = acc[8 + k]
                            stgx[sl] = acc[16 + k]
                            stgn[sl] = acc[24 + k]

                    newacc = []
                    for k in range(8):
                        newacc.append(jnp.where(done, zero, acc[k]))
                    for k in range(8):
                        newacc.append(jnp.where(done, zero, acc[8 + k]))
                    for k in range(8):
                        newacc.append(jnp.where(done, neg, acc[16 + k]))
                    for k in range(8):
                        newacc.append(jnp.where(done, pos, acc[24 + k]))
                    return (n + jnp.where(done, 1, 0), stop) + tuple(newacc)

                st = lax.while_loop(wcond, wbody, carry)

                @pl.when(g + 2 < nb)
                def _():
                    issue(g + 2, bb, bc, semb, semc)
                return st

            carry0 = ((jnp.int32(0), jnp.int32(0)) + tuple([zero] * 8)
                      + tuple([zero] * 8) + tuple([neg] * 8)
                      + tuple([pos] * 8))

            def outer(k, carry):
                carry = half(2 * k, bb0, bc0, semb0, semc0, carry)
                carry = half(2 * k + 1, bb1, bc1, semb1, semc1, carry)
                return carry

            lax.fori_loop(0, nb // 2, outer, carry0)

            osl = pl.ds((t * NW * NPW + vw * HPW) * F, HPW * F)
            ssl = pl.ds(0, HPW * F)
            pltpu.sync_copy(stg1.at[ssl], s1o.at[osl])
            pltpu.sync_copy(stg2.at[ssl], s2o.at[osl])
            pltpu.sync_copy(stgx.at[ssl], mxo.at[osl])
            pltpu.sync_copy(stgn.at[ssl], mno.at[osl])


def _sc_stats(recs, offsi, bs, cs, np_):
    def body(recs, offsi, b0, b1, b2, b3, c0, c1, c2, c3,
             s1o, s2o, mxo, mno, *scratch):
        _stats_body(recs, offsi, (b0, b1, b2, b3), (c0, c1, c2, c3),
                    s1o, s2o, mxo, mno, *scratch)

    fn = pl.kernel(
        body,
        out_type=[jax.ShapeDtypeStruct((4 * np_ * F,), jnp.float32)
                  for _ in range(4)],
        mesh=_mesh,
        scratch_types=[
            pltpu.VMEM((CAP2,), jnp.int32),
            pltpu.VMEM((CAP2,), jnp.int32),
            pltpu.VMEM((GB, F), jnp.float32),
            pltpu.VMEM((GB, F), jnp.float32),
            pltpu.VMEM((GB, F), jnp.float32),
            pltpu.VMEM((GB, F), jnp.float32),
            pltpu.VMEM(((HPW + 8) * F,), jnp.float32),
            pltpu.VMEM(((HPW + 8) * F,), jnp.float32),
            pltpu.VMEM(((HPW + 8) * F,), jnp.float32),
            pltpu.VMEM(((HPW + 8) * F,), jnp.float32),
            pltpu.VMEM((OT,), jnp.int32),
            pltpu.SMEM((OT,), jnp.int32),
            pltpu.SemaphoreType.DMA,
            pltpu.SemaphoreType.DMA,
            pltpu.SemaphoreType.DMA,
            pltpu.SemaphoreType.DMA,
        ],
        compiler_params=_sc_params,
    )
    return fn(recs, offsi, *bs, *cs)


def kernel(x, edge_index, edge_attr, W_in, b_in, We, be, Wpre, bpre, Wpost,
           bpost, Wlin, blin, ln_g, ln_b, Wo1, bo1, Wo2, bo2, Wo3, bo3):
    n = x.shape[0]
    e = edge_index.shape[1]
    L, T = Wpre.shape[0], Wpre.shape[1]
    np_ = NW * NPW

    xp = jnp.zeros((np_, x.shape[1]), jnp.float32).at[:n].set(x)
    h = _mm_gelu(xp, W_in, _row8(b_in), np_)

    recs, offsa = _sc_prep(edge_index[1], edge_index[0])
    offs2 = offsa.reshape(NV, OT)
    deg = (offs2[:, 1:HPW + 1] - offs2[:, :HPW]).reshape(-1).astype(jnp.float32)
    deg_b = jnp.broadcast_to(deg[:, None], (np_, F))

    eap = jnp.pad(edge_attr, ((0, 0), (0, F - edge_attr.shape[1])))

    for l in range(L):
        Wflat = Wpre[l].transpose(1, 0, 2).reshape(3 * F, T * F)
        bflat = bpre[l].reshape(T * F)
        WA = Wflat[:F]
        WB = Wflat[F:2 * F]
        M = We[l] @ Wflat[2 * F:]
        aflat = bflat + be[l] @ Wflat[2 * F:]
        Mp = jnp.zeros((F, 512), jnp.float32).at[:4].set(M)

        ab = _compute_ab(h, WA, WB, _row8(aflat), np_)
        a4, bs = ab[0], ab[1:]
        cs = _compute_c(eap, Mp, e)

        s1f, s2f, mxf, mnf = _sc_stats(recs, offsa, bs, cs, np_)
        s1c = s1f.reshape(4, np_, F)
        s2c = s2f.reshape(4, np_, F)
        mxc = mxf.reshape(4, np_, F)
        mnc = mnf.reshape(4, np_, F)

        WpX = jnp.concatenate([Wpost[l, t, :F, :] for t in range(T)], axis=1)
        W123 = jnp.stack([
            jnp.pad(jnp.concatenate([Wpost[l, t, F:F + 640, :],
                                     Wpost[l, t, F + 640:F + 1280, :],
                                     Wpost[l, t, F + 1280:, :]], axis=1),
                    ((0, 0), (0, 32)))
            for t in range(T)])
        bpost_flat = bpost[l].reshape(-1)
        blinp = bpost_flat @ Wlin[l] + blin[l]

        h = _node_post(h, a4, s1c, s2c, mxc, mnc, deg_b, WpX, W123, Wlin[l],
                       _row8(blinp), _row8(ln_g[l]), _row8(ln_b[l]), np_)

    o = _out_mlp(h, _pad_w(Wo1), _row8(_pad_b(bo1)),
                 _pad_w(Wo2), _row8(_pad_b(bo2)),
                 _pad_w(Wo3), _row8(_pad_b(bo3)), np_)
    return o[:n, 0]


# speedup vs baseline: 53.5520x; 1.0205x over previous
import dataclasses
import functools
import math

import jax
import jax.numpy as jnp
import numpy as np
from jax import lax
from jax.experimental import pallas as pl
from jax.experimental.pallas import tpu as pltpu
from jax.experimental.pallas import tpu_sc as plsc

AVG_LOG = float(np.mean(np.log(np.arange(1, 31, dtype=np.float64))))
NB = 256
F = 128
NEG = -3.4028235e38
POS = 3.4028235e38

NW = 32
NPW = 320
HPW = 160
NV = 64
CAP2 = 3328
ETILE = 2000
GB = 64
OT = 176

_mesh = plsc.VectorSubcoreMesh(core_axis_name="c", subcore_axis_name="s")

_sc_params = pltpu.CompilerParams()
if "needs_layout_passes" in pltpu.CompilerParams.__dataclass_fields__:
    _sc_params = dataclasses.replace(_sc_params, needs_layout_passes=False)


def _gelu(v):
    return 0.5 * v * (1.0 + jax.lax.erf(v * np.float32(1.0 / math.sqrt(2.0))))


def _row8(b):
    return jnp.broadcast_to(b[None, :], (8, b.shape[0]))


def _mm_gelu_body(x_ref, w_ref, b_ref, o_ref):
    o_ref[...] = _gelu(
        jnp.dot(x_ref[...], w_ref[...], preferred_element_type=jnp.float32)
        + b_ref[0, :][None, :])


def _ab_body(h_ref, wa_ref, wb_ref, ba_ref, a_ref, *b_refs):
    h = h_ref[...]
    a = jnp.dot(h, wa_ref[...], preferred_element_type=jnp.float32) + ba_ref[0, :][None, :]
    b = jnp.dot(h, wb_ref[...], preferred_element_type=jnp.float32)
    for c in range(4):
        a_ref[c] = a[:, c * F:(c + 1) * F]
    for c in range(4):
        b_refs[c][...] = b[:, c * F:(c + 1) * F]


def _c_body(ea_ref, m_ref, *c_refs):
    cc = jnp.dot(ea_ref[...], m_ref[...], preferred_element_type=jnp.float32)
    for c in range(4):
        c_refs[c][...] = cc[:, c * F:(c + 1) * F]


def _node_post_body(h_ref, a_ref, s1_ref, s2_ref, mx_ref, mn_ref, deg_ref,
                    wx_ref, w123_ref, wlin_ref, blin_ref, lng_ref, lnb_ref,
                    o_ref):
    t = pl.program_id(1)
    h = h_ref[...]
    deg = deg_ref[...]
    degc = jnp.maximum(deg, 1.0)
    inv = 1.0 / degc
    logd = jnp.log(degc + 1.0)
    s_amp = logd * (1.0 / AVG_LOG)
    s_att = AVG_LOG / logd
    mask = deg > 0.0

    a = a_ref[0]
    s1 = s1_ref[0]
    s2 = s2_ref[0]
    seg_mx = mx_ref[0]
    seg_mn = mn_ref[0]

    s = deg * a + s1
    mean = s * inv
    mx = jnp.where(mask, a + seg_mx, 0.0)
    mn = jnp.where(mask, a + seg_mn, 0.0)
    mean_sq = (deg * a * a + 2.0 * a * s1 + s2) * inv
    var = mean_sq - mean * mean
    std = jnp.sqrt(jnp.maximum(var, 0.0) + 1e-5)

    agg0 = jnp.concatenate([mean, mx, mn, std, var], axis=-1)
    z = jnp.dot(agg0, w123_ref[0], preferred_element_type=jnp.float32)
    out_t = (z[:, 0:32] + s_amp[:, 0:32] * z[:, 32:64]
             + s_att[:, 0:32] * z[:, 64:96])

    @pl.when(t == 0)
    def _():
        o_ref[...] = jnp.dot(h, wx_ref[...], preferred_element_type=jnp.float32)

    for tc in range(4):
        @pl.when(t == tc)
        def _(tc=tc):
            o_ref[:, tc * 32:(tc + 1) * 32] += out_t

    @pl.when(t == 3)
    def _():
        o = o_ref[...]
        o = jnp.dot(o, wlin_ref[...], preferred_element_type=jnp.float32) + blin_ref[0, :][None, :]
        mu = jnp.mean(o, axis=-1, keepdims=True)
        va = jnp.mean((o - mu) * (o - mu), axis=-1, keepdims=True)
        o = (o - mu) / jnp.sqrt(va + 1e-5) * lng_ref[0, :][None, :] + lnb_ref[0, :][None, :]
        o_ref[...] = _gelu(o) + h


def _out_mlp_body(h_ref, w1_ref, b1_ref, w2_ref, b2_ref, w3_ref, b3_ref, o_ref):
    o = _gelu(jnp.dot(h_ref[...], w1_ref[...], preferred_element_type=jnp.float32)
              + b1_ref[0, :][None, :])
    o = _gelu(jnp.dot(o, w2_ref[...], preferred_element_type=jnp.float32)
              + b2_ref[0, :][None, :])
    o_ref[...] = (jnp.dot(o, w3_ref[...], preferred_element_type=jnp.float32)
                  + b3_ref[0, :][None, :])


def _mm_gelu(x, w, b, np_):
    g = np_ // NB
    return pl.pallas_call(
        _mm_gelu_body,
        grid=(g,),
        in_specs=[
            pl.BlockSpec((NB, x.shape[1]), lambda i: (i, 0)),
            pl.BlockSpec(w.shape, lambda i: (0, 0)),
            pl.BlockSpec((8, b.shape[1]), lambda i: (0, 0)),
        ],
        out_specs=pl.BlockSpec((NB, w.shape[1]), lambda i: (i, 0)),
        out_shape=jax.ShapeDtypeStruct((np_, w.shape[1]), jnp.float32),
    )(x, w, b)


def _compute_ab(h, wa, wb, ba, np_):
    g = np_ // NB
    return pl.pallas_call(
        _ab_body,
        grid=(g,),
        in_specs=[
            pl.BlockSpec((NB, F), lambda i: (i, 0)),
            pl.BlockSpec((F, 512), lambda i: (0, 0)),
            pl.BlockSpec((F, 512), lambda i: (0, 0)),
            pl.BlockSpec((8, 512), lambda i: (0, 0)),
        ],
        out_specs=[pl.BlockSpec((4, NB, F), lambda i: (0, i, 0))] + [
            pl.BlockSpec((NB, F), lambda i: (i, 0)) for _ in range(4)],
        out_shape=[jax.ShapeDtypeStruct((4, np_, F), jnp.float32)] + [
            jax.ShapeDtypeStruct((np_, F), jnp.float32) for _ in range(4)],
    )(h, wa, wb, ba)


def _compute_c(eap, mp, e):
    eb = 2000
    g = e // eb
    return pl.pallas_call(
        _c_body,
        grid=(g,),
        in_specs=[
            pl.BlockSpec((eb, F), lambda i: (i, 0)),
            pl.BlockSpec((F, 512), lambda i: (0, 0)),
        ],
        out_specs=[pl.BlockSpec((eb, F), lambda i: (i, 0)) for _ in range(4)],
        out_shape=[jax.ShapeDtypeStruct((e, F), jnp.float32) for _ in range(4)],
    )(eap, mp)


def _node_post(h, a4, s1, s2, mx, mn, deg_b, wx, w123, wlin, blinp, lng, lnb, np_):
    g = np_ // NB
    chunk = lambda i, t: (t, i, 0)
    return pl.pallas_call(
        _node_post_body,
        grid=(g, 4),
        in_specs=[
            pl.BlockSpec((NB, F), lambda i, t: (i, 0)),
            pl.BlockSpec((1, NB, F), chunk),
            pl.BlockSpec((1, NB, F), chunk),
            pl.BlockSpec((1, NB, F), chunk),
            pl.BlockSpec((1, NB, F), chunk),
            pl.BlockSpec((1, NB, F), chunk),
            pl.BlockSpec((NB, F), lambda i, t: (i, 0)),
            pl.BlockSpec((F, F), lambda i, t: (0, 0)),
            pl.BlockSpec((1, 640, F), lambda i, t: (t, 0, 0)),
            pl.BlockSpec((F, F), lambda i, t: (0, 0)),
            pl.BlockSpec((8, F), lambda i, t: (0, 0)),
            pl.BlockSpec((8, F), lambda i, t: (0, 0)),
            pl.BlockSpec((8, F), lambda i, t: (0, 0)),
        ],
        out_specs=pl.BlockSpec((NB, F), lambda i, t: (i, 0)),
        out_shape=jax.ShapeDtypeStruct((np_, F), jnp.float32),
    )(h, a4, s1, s2, mx, mn, deg_b, wx, w123, wlin, blinp, lng, lnb)


def _out_mlp(h, w1, b1, w2, b2, w3, b3, np_):
    g = np_ // NB
    return pl.pallas_call(
        _out_mlp_body,
        grid=(g,),
        in_specs=[pl.BlockSpec((NB, F), lambda i: (i, 0))] + [
            spec for _ in range(3) for spec in (
                pl.BlockSpec((F, F), lambda i: (0, 0)),
                pl.BlockSpec((8, F), lambda i: (0, 0)),
            )
        ],
        out_specs=pl.BlockSpec((NB, F), lambda i: (i, 0)),
        out_shape=jax.ShapeDtypeStruct((np_, F), jnp.float32),
    )(h, w1, b1, w2, b2, w3, b3)


def _pad_w(w, rows=F, cols=F):
    out = jnp.zeros((rows, cols), jnp.float32)
    return out.at[:w.shape[0], :w.shape[1]].set(w)


def _pad_b(b, cols=F):
    out = jnp.zeros((cols,), jnp.float32)
    return out.at[:b.shape[0]].set(b)


def _sc_wid():
    return lax.axis_index("s") * 2 + lax.axis_index("c")


def _dma_wait(src, dst, sem):
    pltpu.make_async_copy(src, dst, sem).wait()


def _prep_body(dsta, srca, recs, offso,
               db0, sb0, db1, sb1,
               ld_u, src_u, eid_u, ld2_u, src2_u, eid2_u,
               ld_s, src_s, eid_s, perm,
               hist, offs, offstg,
               sem0, sem1):
    w = _sc_wid()
    e = dsta.shape[0]
    ntiles = e // ETILE
    iota16 = lax.iota(jnp.int32, 16)
    m0 = iota16 == 0
    npw16 = jnp.full((16,), HPW, jnp.int32)
    zero16i = jnp.zeros((16,), jnp.int32)
    lo = w * NPW
    mid = lo + HPW
    hi = lo + NPW

    pltpu.async_copy(dsta.at[pl.ds(0, ETILE)], db0, sem0)
    pltpu.async_copy(srca.at[pl.ds(0, ETILE)], sb0, sem0)
    pltpu.async_copy(dsta.at[pl.ds(ETILE, ETILE)], db1, sem1)
    pltpu.async_copy(srca.at[pl.ds(ETILE, ETILE)], sb1, sem1)

    def tilework(t, db, sb, sem, carry):
        _dma_wait(dsta.at[pl.ds(0, ETILE)], db, sem)
        _dma_wait(srca.at[pl.ds(0, ETILE)], sb, sem)

        def vec(j, carry):
            off0, off1 = carry
            d = db[pl.ds(j * 16, 16)]
            s = sb[pl.ds(j * 16, 16)]
            eidv = (t * ETILE + j * 16) + iota16
            ma = (d >= lo) & (d < mid)
            mb = (d >= mid) & (d < hi)
            pa = plsc.cumsum(jnp.where(ma, 1, 0).astype(jnp.int32))
            pb = plsc.cumsum(jnp.where(mb, 1, 0).astype(jnp.int32))
            posa = jnp.minimum(off0, CAP2 - 17) + pa - 1
            posb = jnp.minimum(off1, CAP2 - 17) + pb - 1
            plsc.store_scatter(ld_u, [posa], d - lo, mask=ma)
            plsc.store_scatter(src_u, [posa], s, mask=ma)
            plsc.store_scatter(eid_u, [posa], eidv, mask=ma)
            plsc.store_scatter(ld2_u, [posb], d - mid, mask=mb)
            plsc.store_scatter(src2_u, [posb], s, mask=mb)
            plsc.store_scatter(eid2_u, [posb], eidv, mask=mb)
            return (off0 + pa[15], off1 + pb[15])

        carry = lax.fori_loop(0, ETILE // 16, vec, carry)

        @pl.when(t + 2 < ntiles)
        def _():
            pltpu.async_copy(dsta.at[pl.ds((t + 2) * ETILE, ETILE)], db, sem)
            pltpu.async_copy(srca.at[pl.ds((t + 2) * ETILE, ETILE)], sb, sem)
        return carry

    def outer(k, carry):
        carry = tilework(2 * k, db0, sb0, sem0, carry)
        carry = tilework(2 * k + 1, db1, sb1, sem1, carry)
        return carry

    cnt0, cnt1 = lax.fori_loop(0, ntiles // 2, outer,
                               (jnp.int32(0), jnp.int32(0)))

    for v, (vld, vsrc, veid, vcnt) in enumerate(
            [(ld_u, src_u, eid_u, cnt0), (ld2_u, src2_u, eid2_u, cnt1)]):
        vw = w * 2 + v
        ld_u = vld
        src_u = vsrc
        eid_u = veid
        cnt = jnp.minimum(vcnt, CAP2 - 128)
        cnt128 = ((cnt + 127) // 128) * 128

        for j in range(8):
            idxv = cnt + j * 16 + iota16
            mpad = idxv < cnt128
            plsc.store_scatter(ld_u, [idxv], npw16, mask=mpad)
            plsc.store_scatter(src_u, [idxv], zero16i, mask=mpad)
            plsc.store_scatter(eid_u, [idxv], zero16i, mask=mpad)

        def zeroh(i, _):
            hist[i] = jnp.int32(0)
            return 0
        lax.fori_loop(0, OT, zeroh, 0)

        def histb(j, _):
            lv = ld_u[pl.ds(j * 16, 16)]
            for i in range(16):
                l = lv[i]
                hist[l] = hist[l] + 1
            return 0
        lax.fori_loop(0, cnt128 // 16, histb, 0)

        def cumul(b, run):
            hv = hist[b]
            offs[b] = run
            plsc.store_scatter(offstg, [jnp.broadcast_to(b, (16,))],
                               jnp.broadcast_to(run, (16,)), mask=m0)
            return run + hv
        lax.fori_loop(0, OT, cumul, jnp.int32(0))

        def place(j, _):
            lv = ld_u[pl.ds(j * 16, 16)]
            for i in range(16):
                l = lv[i]
                p = offs[l]
                offs[l] = p + 1
                plsc.store_scatter(perm, [jnp.broadcast_to(p, (16,))],
                                   jnp.broadcast_to(j * 16 + i, (16,)), mask=m0)
            return 0
        lax.fori_loop(0, cnt128 // 16, place, 0)

        def apply(j, _):
            idx = perm[pl.ds(j * 16, 16)]
            ld_s[pl.ds(j * 16, 16)] = plsc.load_gather(ld_u, [idx])
            src_s[pl.ds(j * 16, 16)] = plsc.load_gather(src_u, [idx])
            eid_s[pl.ds(j * 16, 16)] = plsc.load_gather(eid_u, [idx])
            return 0
        lax.fori_loop(0, cnt128 // 16, apply, 0)

        pltpu.sync_copy(ld_s, recs.at[pl.ds((0 * NV + vw) * CAP2, CAP2)])
        pltpu.sync_copy(src_s, recs.at[pl.ds((1 * NV + vw) * CAP2, CAP2)])
        pltpu.sync_copy(eid_s, recs.at[pl.ds((2 * NV + vw) * CAP2, CAP2)])
        pltpu.sync_copy(offstg, offso.at[pl.ds(vw * OT, OT)])


def _sc_prep(dsta, srca):
    fn = pl.kernel(
        _prep_body,
        out_type=[
            jax.ShapeDtypeStruct((3 * NV * CAP2,), jnp.int32),
            jax.ShapeDtypeStruct((NV * OT,), jnp.int32),
        ],
        mesh=_mesh,
        scratch_types=[
            pltpu.VMEM((ETILE,), jnp.int32),
            pltpu.VMEM((ETILE,), jnp.int32),
            pltpu.VMEM((ETILE,), jnp.int32),
            pltpu.VMEM((ETILE,), jnp.int32),
            pltpu.VMEM((CAP2,), jnp.int32),
            pltpu.VMEM((CAP2,), jnp.int32),
            pltpu.VMEM((CAP2,), jnp.int32),
            pltpu.VMEM((CAP2,), jnp.int32),
            pltpu.VMEM((CAP2,), jnp.int32),
            pltpu.VMEM((CAP2,), jnp.int32),
            pltpu.VMEM((CAP2,), jnp.int32),
            pltpu.VMEM((CAP2,), jnp.int32),
            pltpu.VMEM((CAP2,), jnp.int32),
            pltpu.VMEM((CAP2,), jnp.int32),
            pltpu.SMEM((OT,), jnp.int32),
            pltpu.SMEM((OT,), jnp.int32),
            pltpu.VMEM((OT,), jnp.int32),
            pltpu.SemaphoreType.DMA,
            pltpu.SemaphoreType.DMA,
        ],
        compiler_params=_sc_params,
    )
    return fn(dsta, srca)


def _stats_body(recs, offsi, bs, cs, s1o, s2o, mxo, mno,
                srcr, eidr, bb0, bc0, bb1, bc1,
                stg1, stg2, stgx, stgn, offv, offm,
                semb0, semc0, semb1, semc1):
    w = _sc_wid()
    zero = jnp.zeros((16,), jnp.float32)
    neg = jnp.full((16,), NEG, jnp.float32)
    pos = jnp.full((16,), POS, jnp.float32)

    for v in range(2):
        vw = 2 * w + v
        pltpu.sync_copy(recs.at[pl.ds((1 * NV + vw) * CAP2, CAP2)], srcr)
        pltpu.sync_copy(recs.at[pl.ds((2 * NV + vw) * CAP2, CAP2)], eidr)
        pltpu.sync_copy(offsi.at[pl.ds(vw * OT, OT)], offv)
        for j in range(OT // 16):
            ov = offv[pl.ds(j * 16, 16)]
            for i in range(16):
                offm[j * 16 + i] = ov[i]
        total = offm[HPW]
        cnt128 = ((total + 127) // 128) * 128
        nb = cnt128 // GB

        for t in range(4):
            bp = bs[t]
            cp = cs[t]

            def initrow(r, _):
                for k in range(8):
                    sl = pl.ds(r * F + k * 16, 16)
                    stg1[sl] = zero
                    stg2[sl] = zero
                    stgx[sl] = neg
                    stgn[sl] = pos
                return 0
            lax.fori_loop(0, HPW + 8, initrow, 0)

            def issue(g, bb, bc, semb, semc):
                idxs = srcr.at[pl.ds(g * GB, GB)]
                pltpu.async_copy(bp.at[idxs], bb, semb)
                idxe = eidr.at[pl.ds(g * GB, GB)]
                pltpu.async_copy(cp.at[idxe], bc, semc)

            @pl.when(nb > 0)
            def _():
                issue(0, bb0, bc0, semb0, semc0)

            @pl.when(nb > 1)
            def _():
                issue(1, bb1, bc1, semb1, semc1)

            def half(g, bb, bc, semb, semc, carry):
                _dma_wait(bp.at[pl.ds(0, GB)], bb, semb)
                _dma_wait(cp.at[pl.ds(0, GB)], bc, semc)
                end = jnp.minimum((g + 1) * GB, total)

                def wcond(st):
                    return st[1] < end

                def wbody(st):
                    n = st[0]
                    r = st[1]
                    acc = st[2:]
                    nxt = offm[n + 1]
                    stop = jnp.minimum(nxt, end)

                    def rec(i, acc):
                        ri = i - g * GB
                        out = []
                        for k in range(8):
                            sl = pl.ds(k * 16, 16)
                            u = bb[ri, sl] + bc[ri, sl]
                            out.append(acc[k] + u)
                            out.append(acc[8 + k] + u * u)
                            out.append(jnp.maximum(acc[16 + k], u))
                            out.append(jnp.minimum(acc[24 + k], u))
                        return tuple(out[0::4] + out[1::4] + out[2::4]
                                     + out[3::4])

                    acc = lax.fori_loop(r, stop, rec, tuple(acc))
                    done = stop == nxt

                    @pl.when(done)
                    def _(acc=acc, n=n):
                        for k in range(8):
                            sl = pl.ds(n * F + k * 16, 16)
                            stg1[sl] = acc[k]
                            stg2[sl] = acc[8 + k]
                            stgx[sl] = acc[16 + k]
                            stgn[sl] = acc[24 + k]

                    newacc = []
                    for k in range(8):
                        newacc.append(jnp.where(done, zero, acc[k]))
                    for k in range(8):
                        newacc.append(jnp.where(done, zero, acc[8 + k]))
                    for k in range(8):
                        newacc.append(jnp.where(done, neg, acc[16 + k]))
                    for k in range(8):
                        newacc.append(jnp.where(done, pos, acc[24 + k]))
                    return (n + jnp.where(done, 1, 0), stop) + tuple(newacc)

                st = lax.while_loop(wcond, wbody, carry)

                @pl.when(g + 2 < nb)
                def _():
                    issue(g + 2, bb, bc, semb, semc)
                return st

            carry0 = ((jnp.int32(0), jnp.int32(0)) + tuple([zero] * 8)
                      + tuple([zero] * 8) + tuple([neg] * 8)
                      + tuple([pos] * 8))

            def outer(k, carry):
                carry = half(2 * k, bb0, bc0, semb0, semc0, carry)
                carry = half(2 * k + 1, bb1, bc1, semb1, semc1, carry)
                return carry

            lax.fori_loop(0, nb // 2, outer, carry0)

            osl = pl.ds((t * NW * NPW + vw * HPW) * F, HPW * F)
            ssl = pl.ds(0, HPW * F)
            pltpu.sync_copy(stg1.at[ssl], s1o.at[osl])
            pltpu.sync_copy(stg2.at[ssl], s2o.at[osl])
            pltpu.sync_copy(stgx.at[ssl], mxo.at[osl])
            pltpu.sync_copy(stgn.at[ssl], mno.at[osl])


def _sc_stats(recs, offsi, bs, cs, np_):
    def body(recs, offsi, b0, b1, b2, b3, c0, c1, c2, c3,
             s1o, s2o, mxo, mno, *scratch):
        _stats_body(recs, offsi, (b0, b1, b2, b3), (c0, c1, c2, c3),
                    s1o, s2o, mxo, mno, *scratch)

    fn = pl.kernel(
        body,
        out_type=[jax.ShapeDtypeStruct((4 * np_ * F,), jnp.float32)
                  for _ in range(4)],
        mesh=_mesh,
        scratch_types=[
            pltpu.VMEM((CAP2,), jnp.int32),
            pltpu.VMEM((CAP2,), jnp.int32),
            pltpu.VMEM((GB, F), jnp.float32),
            pltpu.VMEM((GB, F), jnp.float32),
            pltpu.VMEM((GB, F), jnp.float32),
            pltpu.VMEM((GB, F), jnp.float32),
            pltpu.VMEM(((HPW + 8) * F,), jnp.float32),
            pltpu.VMEM(((HPW + 8) * F,), jnp.float32),
            pltpu.VMEM(((HPW + 8) * F,), jnp.float32),
            pltpu.VMEM(((HPW + 8) * F,), jnp.float32),
            pltpu.VMEM((OT,), jnp.int32),
            pltpu.SMEM((OT,), jnp.int32),
            pltpu.SemaphoreType.DMA,
            pltpu.SemaphoreType.DMA,
            pltpu.SemaphoreType.DMA,
            pltpu.SemaphoreType.DMA,
        ],
        compiler_params=_sc_params,
    )
    return fn(recs, offsi, *bs, *cs)


def kernel(x, edge_index, edge_attr, W_in, b_in, We, be, Wpre, bpre, Wpost,
           bpost, Wlin, blin, ln_g, ln_b, Wo1, bo1, Wo2, bo2, Wo3, bo3):
    n = x.shape[0]
    e = edge_index.shape[1]
    L, T = Wpre.shape[0], Wpre.shape[1]
    np_ = NW * NPW

    xp = jnp.zeros((np_, x.shape[1]), jnp.float32).at[:n].set(x)
    h = _mm_gelu(xp, W_in, _row8(b_in), np_)

    recs, offsa = _sc_prep(edge_index[1], edge_index[0])
    offs2 = offsa.reshape(NV, OT)
    deg = (offs2[:, 1:HPW + 1] - offs2[:, :HPW]).reshape(-1).astype(jnp.float32)
    deg_b = jnp.broadcast_to(deg[:, None], (np_, F))

    eap = jnp.pad(edge_attr, ((0, 0), (0, F - edge_attr.shape[1])))

    for l in range(L):
        Wflat = Wpre[l].transpose(1, 0, 2).reshape(3 * F, T * F)
        bflat = bpre[l].reshape(T * F)
        WA = Wflat[:F]
        WB = Wflat[F:2 * F]
        M = We[l] @ Wflat[2 * F:]
        aflat = bflat + be[l] @ Wflat[2 * F:]
        Mp = jnp.zeros((F, 512), jnp.float32).at[:4].set(M)

        ab = _compute_ab(h, WA, WB, _row8(aflat), np_)
        a4, bs = ab[0], ab[1:]
        cs = _compute_c(eap, Mp, e)

        s1f, s2f, mxf, mnf = _sc_stats(recs, offsa, bs, cs, np_)
        s1c = s1f.reshape(4, np_, F)
        s2c = s2f.reshape(4, np_, F)
        mxc = mxf.reshape(4, np_, F)
        mnc = mnf.reshape(4, np_, F)

        WpX = jnp.concatenate([Wpost[l, t, :F, :] for t in range(T)], axis=1)
        W123 = jnp.stack([
            jnp.pad(jnp.concatenate([Wpost[l, t, F:F + 640, :],
                                     Wpost[l, t, F + 640:F + 1280, :],
                                     Wpost[l, t, F + 1280:, :]], axis=1),
                    ((0, 0), (0, 32)))
            for t in range(T)])
        bpost_flat = bpost[l].reshape(-1)
        blinp = bpost_flat @ Wlin[l] + blin[l]

        h = _node_post(h, a4, s1c, s2c, mxc, mnc, deg_b, WpX, W123, Wlin[l],
                       _row8(blinp), _row8(ln_g[l]), _row8(ln_b[l]), np_)

    o = _out_mlp(h, _pad_w(Wo1), _row8(_pad_b(bo1)),
                 _pad_w(Wo2), _row8(_pad_b(bo2)),
                 _pad_w(Wo3), _row8(_pad_b(bo3)), np_)
    return o[:n, 0]
